# baseline (speedup 1.0000x reference)
import jax
import jax.numpy as jnp
from jax.experimental import pallas as pl
from jax.experimental.pallas import tpu as pltpu

B, L, D = 16, 4096, 1024
NL = 8
LBLK = L // NL
NTILE = D // 128


def _body(mask_ref, feats_ref, out_ref, amax_ref, asum_ref, acnt_ref):
    l = pl.program_id(1)

    @pl.when(l == 0)
    def _():
        amax_ref[...] = jnp.full((1, D), -1e30, jnp.float32)
        asum_ref[...] = jnp.zeros((1, D), jnp.float32)
        acnt_ref[0] = jnp.float32(0.0)

    mf = mask_ref[0].astype(jnp.float32)
    pen = (mf - jnp.float32(1.0)) * jnp.float32(1e30)
    acnt_ref[0] = acnt_ref[0] + jnp.sum(mf) * jnp.float32(1.0 / 128.0)
    for j in range(NTILE):
        js = slice(j * 128, (j + 1) * 128)
        t = feats_ref[0, :, js] * mf
        amax_ref[0, js] = jnp.maximum(amax_ref[0, js],
                                      jnp.max(t + pen, axis=0))
        asum_ref[0, js] = asum_ref[0, js] + jnp.sum(t, axis=0)

    @pl.when(l == NL - 1)
    def _():
        out_ref[0, 0, :D] = amax_ref[0]
        out_ref[0, 0, D:] = asum_ref[0] / acnt_ref[0]


def kernel(feats, mask):
    mask128 = jnp.broadcast_to(
        mask[:, :, None], (B, L, 128)).astype(jnp.int8)
    out = pl.pallas_call(
        _body,
        grid=(B, NL),
        in_specs=[
            pl.BlockSpec((1, LBLK, 128), lambda b, l: (b, l, 0)),
            pl.BlockSpec((1, LBLK, D), lambda b, l: (b, l, 0)),
        ],
        out_specs=pl.BlockSpec((1, 1, 2 * D), lambda b, l: (b, 0, 0)),
        out_shape=jax.ShapeDtypeStruct((B, 1, 2 * D), jnp.float32),
        scratch_shapes=[
            pltpu.VMEM((1, D), jnp.float32),
            pltpu.VMEM((1, D), jnp.float32),
            pltpu.SMEM((1,), jnp.float32),
        ],
    )(mask128, feats)
    return out.reshape(B, 2 * D)


# speedup vs baseline: 1.6289x; 1.6289x over previous
import jax
import jax.numpy as jnp
from jax.experimental import pallas as pl
from jax.experimental.pallas import tpu as pltpu

B, L, D = 16, 4096, 1024
NL = 1
LBLK = L // NL
NTILE = D // 128


def _body(mask_ref, feats_ref, out_ref, amax_ref, asum_ref, acnt_ref):
    l = pl.program_id(1)

    @pl.when(l == 0)
    def _():
        amax_ref[...] = jnp.full((1, D), -1e30, jnp.float32)
        asum_ref[...] = jnp.zeros((1, D), jnp.float32)
        acnt_ref[0] = jnp.float32(0.0)

    mf = mask_ref[0].astype(jnp.float32)
    pen = (mf - jnp.float32(1.0)) * jnp.float32(1e30)
    acnt_ref[0] = acnt_ref[0] + jnp.sum(mf) * jnp.float32(1.0 / 128.0)
    for j in range(NTILE):
        js = slice(j * 128, (j + 1) * 128)
        t = feats_ref[0, :, js] * mf
        amax_ref[0, js] = jnp.maximum(amax_ref[0, js],
                                      jnp.max(t + pen, axis=0))
        asum_ref[0, js] = asum_ref[0, js] + jnp.sum(t, axis=0)

    @pl.when(l == NL - 1)
    def _():
        out_ref[0, 0, :D] = amax_ref[0]
        out_ref[0, 0, D:] = asum_ref[0] / acnt_ref[0]


def kernel(feats, mask):
    mask128 = jnp.broadcast_to(
        mask[:, :, None], (B, L, 128)).astype(jnp.int8)
    out = pl.pallas_call(
        _body,
        grid=(B, NL),
        in_specs=[
            pl.BlockSpec((1, LBLK, 128), lambda b, l: (b, l, 0)),
            pl.BlockSpec((1, LBLK, D), lambda b, l: (b, l, 0)),
        ],
        out_specs=pl.BlockSpec((1, 1, 2 * D), lambda b, l: (b, 0, 0)),
        out_shape=jax.ShapeDtypeStruct((B, 1, 2 * D), jnp.float32),
        scratch_shapes=[
            pltpu.VMEM((1, D), jnp.float32),
            pltpu.VMEM((1, D), jnp.float32),
            pltpu.SMEM((1,), jnp.float32),
        ],
    )(mask128, feats)
    return out.reshape(B, 2 * D)


# speedup vs baseline: 1.6423x; 1.0082x over previous
import jax
import jax.numpy as jnp
from jax.experimental import pallas as pl
from jax.experimental.pallas import tpu as pltpu

B, L, D = 16, 4096, 1024
NSPLIT = 4
LSUB = L // NSPLIT
NTILE = D // 128


def _body(*refs):
    mask_refs = refs[:NSPLIT]
    feats_refs = refs[NSPLIT:2 * NSPLIT]
    out_ref = refs[2 * NSPLIT]

    cnt = jnp.float32(0.0)
    maxs = []
    sums = []
    for s in range(NSPLIT):
        mf = mask_refs[s][0].astype(jnp.float32)
        pen = (mf - jnp.float32(1.0)) * jnp.float32(1e30)
        cnt = cnt + jnp.sum(mf) * jnp.float32(1.0 / 128.0)
        for j in range(NTILE):
            js = slice(j * 128, (j + 1) * 128)
            t = feats_refs[s][0, :, js] * mf
            bmax = jnp.max(t + pen, axis=0)
            bsum = jnp.sum(t, axis=0)
            if s == 0:
                maxs.append(bmax)
                sums.append(bsum)
            else:
                maxs[j] = jnp.maximum(maxs[j], bmax)
                sums[j] = sums[j] + bsum
    inv = jnp.float32(1.0) / cnt
    for j in range(NTILE):
        js = slice(j * 128, (j + 1) * 128)
        out_ref[0, 0, js] = maxs[j]
        out_ref[0, 0, D + j * 128:D + (j + 1) * 128] = sums[j] * inv


def kernel(feats, mask):
    mask128 = jnp.broadcast_to(
        mask[:, :, None], (B, L, 128)).astype(jnp.int8)
    in_specs = [
        pl.BlockSpec((1, LSUB, 128), (lambda b, s=s: (b, s, 0)))
        for s in range(NSPLIT)
    ] + [
        pl.BlockSpec((1, LSUB, D), (lambda b, s=s: (b, s, 0)))
        for s in range(NSPLIT)
    ]
    out = pl.pallas_call(
        _body,
        grid=(B,),
        in_specs=in_specs,
        out_specs=pl.BlockSpec((1, 1, 2 * D), lambda b: (b, 0, 0)),
        out_shape=jax.ShapeDtypeStruct((B, 1, 2 * D), jnp.float32),
    )(*([mask128] * NSPLIT + [feats] * NSPLIT))
    return out.reshape(B, 2 * D)
